# baseline (device time: 15265 ns/iter reference)
import jax
import jax.numpy as jnp
from jax import lax
from jax.experimental import pallas as pl
from jax.experimental.pallas import tpu as pltpu

T = 256
D = 512
V_LOCAL = 4096
CHUNK = 512
N_CHUNKS = V_LOCAL // CHUNK


def kernel(x, W, labels):
    def body(x_ref, w_ref, labels_ref, out_ref,
             s_acc, ll_acc, comm_ref, send_sem, recv_sem):
        i = pl.program_id(0)
        my_x = lax.axis_index("x")
        my_y = lax.axis_index("y")
        my_z = lax.axis_index("z")
        partner = (my_x, 1 - my_y, my_z)
        barrier_sem = pltpu.get_barrier_semaphore()

        @pl.when(i == 0)
        def _():
            pl.semaphore_signal(
                barrier_sem, inc=1,
                device_id=partner, device_id_type=pl.DeviceIdType.MESH,
            )
            s_acc[0, :] = jnp.zeros((T,), jnp.float32)
            ll_acc[0, :] = jnp.zeros((T,), jnp.float32)

        logits = jnp.dot(
            x_ref[:, :], w_ref[:, :], preferred_element_type=jnp.float32
        )
        s_acc[0, :] += jnp.sum(jnp.exp(logits), axis=1)
        local_idx = labels_ref[:] - (my_y * V_LOCAL + i * CHUNK)
        col = lax.broadcasted_iota(jnp.int32, (T, CHUNK), 1)
        ll_acc[0, :] += jnp.sum(
            jnp.where(col == local_idx[:, None], logits, 0.0), axis=1
        )

        @pl.when(i == N_CHUNKS - 1)
        def _():
            comm_ref[0, 0, :] = s_acc[0, :]
            comm_ref[0, 1, :] = ll_acc[0, :]
            pl.semaphore_wait(barrier_sem, 1)
            rdma = pltpu.make_async_remote_copy(
                src_ref=comm_ref.at[0],
                dst_ref=comm_ref.at[1],
                send_sem=send_sem,
                recv_sem=recv_sem,
                device_id=partner,
                device_id_type=pl.DeviceIdType.MESH,
            )
            rdma.start()
            rdma.wait()
            s_r = comm_ref[1, 0, :]
            ll_r = comm_ref[1, 1, :]
            out_ref[:] = jnp.log(s_acc[0, :] + s_r) - (ll_acc[0, :] + ll_r)

    return pl.pallas_call(
        body,
        grid=(N_CHUNKS,),
        out_shape=jax.ShapeDtypeStruct((T,), jnp.float32),
        in_specs=[
            pl.BlockSpec((T, D), lambda i: (0, 0)),
            pl.BlockSpec((D, CHUNK), lambda i: (0, i)),
            pl.BlockSpec((T,), lambda i: (0,)),
        ],
        out_specs=pl.BlockSpec((T,), lambda i: (0,)),
        scratch_shapes=[
            pltpu.VMEM((1, T), jnp.float32),
            pltpu.VMEM((1, T), jnp.float32),
            pltpu.VMEM((2, 2, T), jnp.float32),
            pltpu.SemaphoreType.DMA,
            pltpu.SemaphoreType.DMA,
        ],
        compiler_params=pltpu.CompilerParams(
            dimension_semantics=("arbitrary",),
            collective_id=0,
        ),
    )(x, W, labels)


# device time: 14524 ns/iter; 1.0510x vs baseline; 1.0510x over previous
import jax
import jax.numpy as jnp
from jax import lax
from jax.experimental import pallas as pl
from jax.experimental.pallas import tpu as pltpu

T = 256
D = 512
V_LOCAL = 4096
N_CHUNKS = 4
CHUNK = V_LOCAL // N_CHUNKS


def kernel(x, W, labels):
    def body(x_hbm, w_hbm, labels_hbm, out_ref,
             x_v, labels_v, w_v, comm_ref,
             send_sem, recv_sem, sem_x, sem_l, sem_w):
        my_x = lax.axis_index("x")
        my_y = lax.axis_index("y")
        my_z = lax.axis_index("z")
        partner = (my_x, 1 - my_y, my_z)

        barrier_sem = pltpu.get_barrier_semaphore()
        pl.semaphore_signal(
            barrier_sem, inc=1,
            device_id=partner, device_id_type=pl.DeviceIdType.MESH,
        )

        cp_x = pltpu.make_async_copy(x_hbm, x_v, sem_x)
        cp_x.start()
        cp_l = pltpu.make_async_copy(labels_hbm, labels_v, sem_l)
        cp_l.start()
        cp_w = []
        for k in range(N_CHUNKS):
            cp = pltpu.make_async_copy(
                w_hbm.at[:, pl.ds(k * CHUNK, CHUNK)], w_v.at[k], sem_w.at[k]
            )
            cp.start()
            cp_w.append(cp)

        cp_x.wait()
        cp_l.wait()
        xv = x_v[:, :]
        lbl = labels_v[:]

        s = jnp.zeros((T,), jnp.float32)
        ll = jnp.zeros((T,), jnp.float32)
        for k in range(N_CHUNKS):
            cp_w[k].wait()
            logits = jnp.dot(
                xv, w_v[k], preferred_element_type=jnp.float32
            )
            s = s + jnp.sum(jnp.exp(logits), axis=1)
            local_idx = lbl - (my_y * V_LOCAL + k * CHUNK)
            col = lax.broadcasted_iota(jnp.int32, (T, CHUNK), 1)
            ll = ll + jnp.sum(
                jnp.where(col == local_idx[:, None], logits, 0.0), axis=1
            )

        comm_ref[0, 0, :] = s
        comm_ref[0, 1, :] = ll
        pl.semaphore_wait(barrier_sem, 1)
        rdma = pltpu.make_async_remote_copy(
            src_ref=comm_ref.at[0],
            dst_ref=comm_ref.at[1],
            send_sem=send_sem,
            recv_sem=recv_sem,
            device_id=partner,
            device_id_type=pl.DeviceIdType.MESH,
        )
        rdma.start()
        rdma.wait()
        out_ref[:] = jnp.log(s + comm_ref[1, 0, :]) - (ll + comm_ref[1, 1, :])

    return pl.pallas_call(
        body,
        out_shape=jax.ShapeDtypeStruct((T,), jnp.float32),
        in_specs=[
            pl.BlockSpec(memory_space=pltpu.MemorySpace.HBM),
            pl.BlockSpec(memory_space=pltpu.MemorySpace.HBM),
            pl.BlockSpec(memory_space=pltpu.MemorySpace.HBM),
        ],
        out_specs=pl.BlockSpec(memory_space=pltpu.VMEM),
        scratch_shapes=[
            pltpu.VMEM((T, D), jnp.float32),
            pltpu.VMEM((T,), jnp.int32),
            pltpu.VMEM((N_CHUNKS, D, CHUNK), jnp.float32),
            pltpu.VMEM((2, 2, T), jnp.float32),
            pltpu.SemaphoreType.DMA,
            pltpu.SemaphoreType.DMA,
            pltpu.SemaphoreType.DMA,
            pltpu.SemaphoreType.DMA,
            pltpu.SemaphoreType.DMA((N_CHUNKS,)),
        ],
        compiler_params=pltpu.CompilerParams(collective_id=0),
    )(x, W, labels)


# device time: 10503 ns/iter; 1.4534x vs baseline; 1.3828x over previous
import jax
import jax.numpy as jnp
from jax import lax
from jax.experimental import pallas as pl
from jax.experimental.pallas import tpu as pltpu

T = 256
D = 512
V_LOCAL = 4096
N_CHUNKS = 4
CHUNK = V_LOCAL // N_CHUNKS


def kernel(x, W, labels):
    x = pltpu.with_memory_space_constraint(x, pltpu.MemorySpace.HBM)
    W = pltpu.with_memory_space_constraint(W, pltpu.MemorySpace.HBM)
    labels = pltpu.with_memory_space_constraint(labels, pltpu.MemorySpace.HBM)

    def body(x_hbm, w_hbm, labels_hbm, out_ref,
             x_v, labels_v, w_v, comm_ref,
             send_sem, recv_sem, sem_x, sem_l, sem_w):
        my_x = lax.axis_index("x")
        my_y = lax.axis_index("y")
        my_z = lax.axis_index("z")
        partner = (my_x, 1 - my_y, my_z)

        barrier_sem = pltpu.get_barrier_semaphore()
        pl.semaphore_signal(
            barrier_sem, inc=1,
            device_id=partner, device_id_type=pl.DeviceIdType.MESH,
        )

        cp_x = pltpu.make_async_copy(x_hbm, x_v, sem_x)
        cp_x.start()
        cp_l = pltpu.make_async_copy(labels_hbm, labels_v, sem_l)
        cp_l.start()
        cp_w = []
        for k in range(N_CHUNKS):
            cp = pltpu.make_async_copy(
                w_hbm.at[:, pl.ds(k * CHUNK, CHUNK)], w_v.at[k], sem_w.at[k]
            )
            cp.start()
            cp_w.append(cp)

        cp_x.wait()
        cp_l.wait()
        xv = x_v[:, :]
        lbl = labels_v[:]

        s = jnp.zeros((T,), jnp.float32)
        ll = jnp.zeros((T,), jnp.float32)
        for k in range(N_CHUNKS):
            cp_w[k].wait()
            logits = jnp.dot(
                xv, w_v[k], preferred_element_type=jnp.float32
            )
            s = s + jnp.sum(jnp.exp(logits), axis=1)
            local_idx = lbl - (my_y * V_LOCAL + k * CHUNK)
            col = lax.broadcasted_iota(jnp.int32, (T, CHUNK), 1)
            ll = ll + jnp.sum(
                jnp.where(col == local_idx[:, None], logits, 0.0), axis=1
            )

        comm_ref[0, 0, :] = s
        comm_ref[0, 1, :] = ll
        pl.semaphore_wait(barrier_sem, 1)
        rdma = pltpu.make_async_remote_copy(
            src_ref=comm_ref.at[0],
            dst_ref=comm_ref.at[1],
            send_sem=send_sem,
            recv_sem=recv_sem,
            device_id=partner,
            device_id_type=pl.DeviceIdType.MESH,
        )
        rdma.start()
        rdma.wait()
        out_ref[:] = jnp.log(s + comm_ref[1, 0, :]) - (ll + comm_ref[1, 1, :])

    return pl.pallas_call(
        body,
        out_shape=jax.ShapeDtypeStruct((T,), jnp.float32),
        in_specs=[
            pl.BlockSpec(memory_space=pltpu.MemorySpace.HBM),
            pl.BlockSpec(memory_space=pltpu.MemorySpace.HBM),
            pl.BlockSpec(memory_space=pltpu.MemorySpace.HBM),
        ],
        out_specs=pl.BlockSpec(memory_space=pltpu.VMEM),
        scratch_shapes=[
            pltpu.VMEM((T, D), jnp.float32),
            pltpu.VMEM((T,), jnp.int32),
            pltpu.VMEM((N_CHUNKS, D, CHUNK), jnp.float32),
            pltpu.VMEM((2, 2, T), jnp.float32),
            pltpu.SemaphoreType.DMA,
            pltpu.SemaphoreType.DMA,
            pltpu.SemaphoreType.DMA,
            pltpu.SemaphoreType.DMA,
            pltpu.SemaphoreType.DMA((N_CHUNKS,)),
        ],
        compiler_params=pltpu.CompilerParams(collective_id=0),
    )(x, W, labels)
